# baseline (speedup 1.0000x reference)
import jax
import jax.numpy as jnp
from jax import lax
from jax.experimental import pallas as pl
from jax.experimental.pallas import tpu as pltpu
from jax.experimental.pallas import tpu_sc as plsc

RES = 128
N_RAYS = 65536
N_SAMPLES = 128
NUM_CORES = 2
NUM_SUBCORES = 16
NW = NUM_CORES * NUM_SUBCORES
RPW = N_RAYS // NW
CHUNK = 16
NCHUNK = RPW // CHUNK
NWORDS = RES ** 3 // 32
WPS = NWORDS // NUM_SUBCORES
NGROUP = N_SAMPLES // 16
ROW = 2 * N_SAMPLES
INV_N = 1.0 / N_SAMPLES


def _body(rays_o_hbm, rays_d_hbm, occs_hbm, out_hbm,
          rays_ov, rays_dv, stage_v, packed_v, bitmask_v,
          setupf, setupi, outbuf_a, outbuf_b, spmem, sem_a, sem_b):
    cid = lax.axis_index("c")
    sid = lax.axis_index("s")
    wid = sid * NUM_CORES + cid
    ray0 = wid * RPW

    for c in range(3):
        pltpu.sync_copy(rays_o_hbm.at[pl.ds(c * N_RAYS + ray0, RPW)],
                        rays_ov.at[pl.ds(c * RPW, RPW)])
        pltpu.sync_copy(rays_d_hbm.at[pl.ds(c * N_RAYS + ray0, RPW)],
                        rays_dv.at[pl.ds(c * RPW, RPW)])

    one16 = jnp.ones((16,), jnp.int32)
    zero16i = jnp.zeros((16,), jnp.int32)

    for b in range(32):
        pltpu.sync_copy(occs_hbm.at[pl.ds(b * NWORDS + sid * WPS, WPS)],
                        stage_v)

        if b == 0:
            @plsc.parallel_loop(0, WPS // 16, unroll=2)
            def _pk0(w16):
                off = w16 * 16
                v = stage_v[pl.ds(off, 16)]
                packed_v[pl.ds(off, 16)] = jnp.where(v > 0.5, one16, zero16i)
        else:
            @plsc.parallel_loop(0, WPS // 16, unroll=2)
            def _pk(w16):
                off = w16 * 16
                v = stage_v[pl.ds(off, 16)]
                bit = jnp.where(v > 0.5, one16, zero16i) << b
                packed_v[pl.ds(off, 16)] = packed_v[pl.ds(off, 16)] | bit

    pltpu.sync_copy(packed_v, spmem.at[pl.ds(sid * WPS, WPS)])
    plsc.subcore_barrier()
    pltpu.sync_copy(spmem, bitmask_v)

    iota_f = lax.iota(jnp.int32, 16).astype(jnp.float32)

    def compute_chunk(ch, outbuf):
        c16 = ch * CHUNK
        ox = rays_ov[pl.ds(c16, 16)]
        oy = rays_ov[pl.ds(RPW + c16, 16)]
        oz = rays_ov[pl.ds(2 * RPW + c16, 16)]
        dx = rays_dv[pl.ds(c16, 16)]
        dy = rays_dv[pl.ds(RPW + c16, 16)]
        dz = rays_dv[pl.ds(2 * RPW + c16, 16)]

        eps = jnp.float32(1e-8)
        dsx = jnp.where(jnp.abs(dx) < eps, eps, dx)
        dsy = jnp.where(jnp.abs(dy) < eps, eps, dy)
        dsz = jnp.where(jnp.abs(dz) < eps, eps, dz)
        t1x = (-1.0 - ox) / dsx
        t2x = (1.0 - ox) / dsx
        t1y = (-1.0 - oy) / dsy
        t2y = (1.0 - oy) / dsy
        t1z = (-1.0 - oz) / dsz
        t2z = (1.0 - oz) / dsz
        tmin = jnp.maximum(jnp.maximum(jnp.minimum(t1x, t2x),
                                       jnp.minimum(t1y, t2y)),
                           jnp.minimum(t1z, t2z))
        tmax = jnp.minimum(jnp.minimum(jnp.maximum(t1x, t2x),
                                       jnp.maximum(t1y, t2y)),
                           jnp.maximum(t1z, t2z))
        tmin = jnp.minimum(jnp.maximum(tmin, 0.0), 1e10)
        tmax = jnp.minimum(jnp.maximum(tmax, 0.0), 1e10)
        valid1 = jnp.where(tmax > tmin, one16, zero16i)
        span = tmax - tmin
        setupf[pl.ds(0, 16)] = tmin
        setupf[pl.ds(16, 16)] = span
        setupi[pl.ds(0, 16)] = valid1

        @plsc.parallel_loop(0, CHUNK, unroll=8)
        def _ray(r):
            tmin_s = setupf[pl.ds(r, 16)][0]
            span_s = setupf[pl.ds(16 + r, 16)][0]
            valid_s = setupi[pl.ds(r, 16)][0]
            ox_s = rays_ov[pl.ds(c16 + r, 16)][0]
            oy_s = rays_ov[pl.ds(RPW + c16 + r, 16)][0]
            oz_s = rays_ov[pl.ds(2 * RPW + c16 + r, 16)][0]
            dx_s = rays_dv[pl.ds(c16 + r, 16)][0]
            dy_s = rays_dv[pl.ds(RPW + c16 + r, 16)][0]
            dz_s = rays_dv[pl.ds(2 * RPW + c16 + r, 16)][0]
            roff = r * ROW
            for g in range(NGROUP):
                jf = iota_f + float(g * 16)
                ts = tmin_s + (jf * INV_N) * span_s
                te = tmin_s + ((jf + 1.0) * INV_N) * span_s
                tm = 0.5 * (ts + te)
                fx = jnp.minimum(jnp.maximum((ox_s + tm * dx_s + 1.0) * 64.0,
                                             0.0), 127.0)
                fy = jnp.minimum(jnp.maximum((oy_s + tm * dy_s + 1.0) * 64.0,
                                             0.0), 127.0)
                fz = jnp.minimum(jnp.maximum((oz_s + tm * dz_s + 1.0) * 64.0,
                                             0.0), 127.0)
                ix = fx.astype(jnp.int32)
                iy = fy.astype(jnp.int32)
                iz = fz.astype(jnp.int32)
                flat = ((ix << 7) | iy) << 7 | iz
                word = flat & 0xFFFF
                bpl = flat >> 16
                w = plsc.load_gather(bitmask_v, [word])
                m = ((w >> bpl) & valid_s) != 0
                outbuf[pl.ds(roff + g * 16, 16)] = jnp.where(m, ts, 0.0)
                outbuf[pl.ds(roff + N_SAMPLES + g * 16, 16)] = (
                    jnp.where(m, te, 0.0))

    def pair_body(p, carry):
        ch_a = 2 * p
        ch_b = 2 * p + 1
        blk = CHUNK * ROW

        @pl.when(p > 0)
        def _():
            pltpu.make_async_copy(
                outbuf_a, out_hbm.at[pl.ds(0, blk)], sem_a).wait()

        compute_chunk(ch_a, outbuf_a)
        pltpu.async_copy(
            outbuf_a, out_hbm.at[pl.ds((ray0 + ch_a * CHUNK) * ROW, blk)],
            sem_a)

        @pl.when(p > 0)
        def _():
            pltpu.make_async_copy(
                outbuf_b, out_hbm.at[pl.ds(0, blk)], sem_b).wait()

        compute_chunk(ch_b, outbuf_b)
        pltpu.async_copy(
            outbuf_b, out_hbm.at[pl.ds((ray0 + ch_b * CHUNK) * ROW, blk)],
            sem_b)
        return carry

    lax.fori_loop(0, NCHUNK // 2, pair_body, 0)
    blk = CHUNK * ROW
    pltpu.make_async_copy(outbuf_a, out_hbm.at[pl.ds(0, blk)], sem_a).wait()
    pltpu.make_async_copy(outbuf_b, out_hbm.at[pl.ds(0, blk)], sem_b).wait()


@jax.jit
def kernel(rays_o, rays_d, occs):
    mesh = plsc.VectorSubcoreMesh(core_axis_name="c", subcore_axis_name="s")
    out = pl.kernel(
        _body,
        out_type=jax.ShapeDtypeStruct((N_RAYS * ROW,), jnp.float32),
        mesh=mesh,
        scratch_types=[
            pltpu.VMEM((3 * RPW + 16,), jnp.float32),
            pltpu.VMEM((3 * RPW + 16,), jnp.float32),
            pltpu.VMEM((WPS,), jnp.float32),
            pltpu.VMEM((WPS,), jnp.int32),
            pltpu.VMEM((NWORDS,), jnp.int32),
            pltpu.VMEM((48,), jnp.float32),
            pltpu.VMEM((32,), jnp.int32),
            pltpu.VMEM((CHUNK * ROW,), jnp.float32),
            pltpu.VMEM((CHUNK * ROW,), jnp.float32),
            pltpu.VMEM_SHARED((NWORDS,), jnp.int32),
            pltpu.SemaphoreType.DMA,
            pltpu.SemaphoreType.DMA,
        ],
        compiler_params=pltpu.CompilerParams(needs_layout_passes=False),
    )(rays_o.T.reshape(-1), rays_d.T.reshape(-1), occs)
    return out.reshape(N_RAYS, 2, N_SAMPLES).transpose(0, 2, 1)


# speedup vs baseline: 1.7088x; 1.7088x over previous
import jax
import jax.numpy as jnp
from jax import lax
from jax.experimental import pallas as pl
from jax.experimental.pallas import tpu as pltpu
from jax.experimental.pallas import tpu_sc as plsc

RES = 128
N_RAYS = 65536
N_SAMPLES = 128
NUM_CORES = 2
NUM_SUBCORES = 16
NW = NUM_CORES * NUM_SUBCORES
RPW = N_RAYS // NW
CHUNK = 16
NCHUNK = RPW // CHUNK
NWORDS = RES ** 3 // 32
WPS = NWORDS // NUM_SUBCORES
NGROUP = N_SAMPLES // 16
ROW = 2 * N_SAMPLES
INV_N = 1.0 / N_SAMPLES


def _body(rays_o_hbm, rays_d_hbm, occs_hbm, out_hbm,
          rays_ov, rays_dv, stage_v, packed_v, bitmask_v,
          setupf, setupi, outbuf_a, outbuf_b, spmem, sem_a, sem_b):
    cid = lax.axis_index("c")
    sid = lax.axis_index("s")
    wid = sid * NUM_CORES + cid
    ray0 = wid * RPW

    for c in range(3):
        pltpu.sync_copy(rays_o_hbm.at[pl.ds(c * N_RAYS + ray0, RPW)],
                        rays_ov.at[pl.ds(c * RPW, RPW)])
        pltpu.sync_copy(rays_d_hbm.at[pl.ds(c * N_RAYS + ray0, RPW)],
                        rays_dv.at[pl.ds(c * RPW, RPW)])

    one16 = jnp.ones((16,), jnp.int32)
    zero16i = jnp.zeros((16,), jnp.int32)

    PBLK = 512
    for k in range(WPS // PBLK):
        for b in range(32):
            pltpu.async_copy(
                occs_hbm.at[pl.ds(b * NWORDS + sid * WPS + k * PBLK, PBLK)],
                stage_v.at[pl.ds(b * PBLK, PBLK)], sem_a)
        for b in range(32):
            pltpu.make_async_copy(
                occs_hbm.at[pl.ds(0, PBLK)], stage_v.at[pl.ds(0, PBLK)],
                sem_a).wait()

        @plsc.parallel_loop(0, PBLK // 16, unroll=2)
        def _pk(w16):
            off = w16 * 16
            acc = zero16i
            for b in range(32):
                v = stage_v[pl.ds(b * PBLK + off, 16)]
                acc = acc | (jnp.where(v > 0.5, one16, zero16i) << b)
            packed_v[pl.ds(k * PBLK + off, 16)] = acc

    pltpu.sync_copy(packed_v, spmem.at[pl.ds(sid * WPS, WPS)])
    plsc.subcore_barrier()
    pltpu.sync_copy(spmem, bitmask_v)

    iota_f = lax.iota(jnp.int32, 16).astype(jnp.float32)

    def compute_chunk(ch, outbuf):
        c16 = ch * CHUNK
        ox = rays_ov[pl.ds(c16, 16)]
        oy = rays_ov[pl.ds(RPW + c16, 16)]
        oz = rays_ov[pl.ds(2 * RPW + c16, 16)]
        dx = rays_dv[pl.ds(c16, 16)]
        dy = rays_dv[pl.ds(RPW + c16, 16)]
        dz = rays_dv[pl.ds(2 * RPW + c16, 16)]

        eps = jnp.float32(1e-8)
        dsx = jnp.where(jnp.abs(dx) < eps, eps, dx)
        dsy = jnp.where(jnp.abs(dy) < eps, eps, dy)
        dsz = jnp.where(jnp.abs(dz) < eps, eps, dz)
        t1x = (-1.0 - ox) / dsx
        t2x = (1.0 - ox) / dsx
        t1y = (-1.0 - oy) / dsy
        t2y = (1.0 - oy) / dsy
        t1z = (-1.0 - oz) / dsz
        t2z = (1.0 - oz) / dsz
        tmin = jnp.maximum(jnp.maximum(jnp.minimum(t1x, t2x),
                                       jnp.minimum(t1y, t2y)),
                           jnp.minimum(t1z, t2z))
        tmax = jnp.minimum(jnp.minimum(jnp.maximum(t1x, t2x),
                                       jnp.maximum(t1y, t2y)),
                           jnp.maximum(t1z, t2z))
        tmin = jnp.minimum(jnp.maximum(tmin, 0.0), 1e10)
        tmax = jnp.minimum(jnp.maximum(tmax, 0.0), 1e10)
        valid1 = jnp.where(tmax > tmin, one16, zero16i)
        span = tmax - tmin
        setupf[pl.ds(0, 16)] = tmin
        setupf[pl.ds(16, 16)] = span
        setupi[pl.ds(0, 16)] = valid1

        @plsc.parallel_loop(0, CHUNK, unroll=4)
        def _ray(r):
            tmin_s = setupf[pl.ds(r, 16)][0]
            span_s = setupf[pl.ds(16 + r, 16)][0]
            valid_s = setupi[pl.ds(r, 16)][0]
            ox_s = rays_ov[pl.ds(c16 + r, 16)][0]
            oy_s = rays_ov[pl.ds(RPW + c16 + r, 16)][0]
            oz_s = rays_ov[pl.ds(2 * RPW + c16 + r, 16)][0]
            dx_s = rays_dv[pl.ds(c16 + r, 16)][0]
            dy_s = rays_dv[pl.ds(RPW + c16 + r, 16)][0]
            dz_s = rays_dv[pl.ds(2 * RPW + c16 + r, 16)][0]
            roff = r * ROW
            for g in range(NGROUP):
                jf = iota_f + float(g * 16)
                ts = tmin_s + (jf * INV_N) * span_s
                te = tmin_s + ((jf + 1.0) * INV_N) * span_s
                tm = 0.5 * (ts + te)
                fx = jnp.minimum(jnp.maximum((ox_s + tm * dx_s + 1.0) * 64.0,
                                             0.0), 127.0)
                fy = jnp.minimum(jnp.maximum((oy_s + tm * dy_s + 1.0) * 64.0,
                                             0.0), 127.0)
                fz = jnp.minimum(jnp.maximum((oz_s + tm * dz_s + 1.0) * 64.0,
                                             0.0), 127.0)
                ix = fx.astype(jnp.int32)
                iy = fy.astype(jnp.int32)
                iz = fz.astype(jnp.int32)
                flat = ((ix << 7) | iy) << 7 | iz
                word = flat & 0xFFFF
                bpl = flat >> 16
                w = plsc.load_gather(bitmask_v, [word])
                m = ((w >> bpl) & valid_s) != 0
                outbuf[pl.ds(roff + g * 16, 16)] = jnp.where(m, ts, 0.0)
                outbuf[pl.ds(roff + N_SAMPLES + g * 16, 16)] = (
                    jnp.where(m, te, 0.0))

    def pair_body(p, carry):
        ch_a = 2 * p
        ch_b = 2 * p + 1
        blk = CHUNK * ROW

        @pl.when(p > 0)
        def _():
            pltpu.make_async_copy(
                outbuf_a, out_hbm.at[pl.ds(0, blk)], sem_a).wait()

        compute_chunk(ch_a, outbuf_a)
        pltpu.async_copy(
            outbuf_a, out_hbm.at[pl.ds((ray0 + ch_a * CHUNK) * ROW, blk)],
            sem_a)

        @pl.when(p > 0)
        def _():
            pltpu.make_async_copy(
                outbuf_b, out_hbm.at[pl.ds(0, blk)], sem_b).wait()

        compute_chunk(ch_b, outbuf_b)
        pltpu.async_copy(
            outbuf_b, out_hbm.at[pl.ds((ray0 + ch_b * CHUNK) * ROW, blk)],
            sem_b)
        return carry

    lax.fori_loop(0, NCHUNK // 2, pair_body, 0)
    blk = CHUNK * ROW
    pltpu.make_async_copy(outbuf_a, out_hbm.at[pl.ds(0, blk)], sem_a).wait()
    pltpu.make_async_copy(outbuf_b, out_hbm.at[pl.ds(0, blk)], sem_b).wait()


@jax.jit
def kernel(rays_o, rays_d, occs):
    mesh = plsc.VectorSubcoreMesh(core_axis_name="c", subcore_axis_name="s")
    out = pl.kernel(
        _body,
        out_type=jax.ShapeDtypeStruct((N_RAYS * ROW,), jnp.float32),
        mesh=mesh,
        scratch_types=[
            pltpu.VMEM((3 * RPW + 16,), jnp.float32),
            pltpu.VMEM((3 * RPW + 16,), jnp.float32),
            pltpu.VMEM((32 * 512,), jnp.float32),
            pltpu.VMEM((WPS,), jnp.int32),
            pltpu.VMEM((NWORDS,), jnp.int32),
            pltpu.VMEM((48,), jnp.float32),
            pltpu.VMEM((32,), jnp.int32),
            pltpu.VMEM((CHUNK * ROW,), jnp.float32),
            pltpu.VMEM((CHUNK * ROW,), jnp.float32),
            pltpu.VMEM_SHARED((NWORDS,), jnp.int32),
            pltpu.SemaphoreType.DMA,
            pltpu.SemaphoreType.DMA,
        ],
        compiler_params=pltpu.CompilerParams(needs_layout_passes=False),
    )(rays_o.T.reshape(-1), rays_d.T.reshape(-1), occs)
    return out.reshape(N_RAYS, 2, N_SAMPLES).transpose(0, 2, 1)
